# baseline (speedup 1.0000x reference)
import functools

import jax
import jax.numpy as jnp
from jax import lax
from jax.experimental import pallas as pl
from jax.experimental.pallas import tpu as pltpu
from jax.experimental.pallas import tpu_sc as plsc

N = 10000
E = 160000
F = 256
H = 128
NC = 2
NS = 16
CH = 128
EPW = 10240
E_PAD = NS * EPW
NCH = EPW // CH
CPS = 40
N_PAD = 10240
RPS = N_PAD // NS
DEGW = 8
BLK = 1000


def _sc_agg_body(with_deg, *refs):
    if with_deg:
        (src_hbm, dst_hbm, table_hbm, out_hbm, deg_hbm,
         src_v, dst_v, buf0, buf1, ones_v, dz_v, acc, dacc,
         gsem0, gsem1, ssem0, ssem1, dsem) = refs
    else:
        (src_hbm, dst_hbm, table_hbm, out_hbm,
         src_v, dst_v, buf0, buf1, acc,
         gsem0, gsem1, ssem0, ssem1) = refs

    c = lax.axis_index("c")
    s = lax.axis_index("s")

    zero16 = jnp.zeros((16,), jnp.float32)

    def _zb(i, _):
        buf0[i // 8, pl.ds((i % 8) * 16, 16)] = zero16
        return _

    lax.fori_loop(0, (CH * H) // 16, _zb, None)
    for k in range(RPS // CH):
        pltpu.sync_copy(buf0, acc.at[pl.ds(s * RPS + k * CH, CH)])

    if with_deg:
        one16 = jnp.ones((16,), jnp.float32)

        def _ob(i, _):
            ones_v[pl.ds(i * 16, 16)] = one16
            return _

        lax.fori_loop(0, CH // 16, _ob, None)

        def _dz(i, _):
            dz_v[pl.ds(i * 16, 16)] = zero16
            return _

        lax.fori_loop(0, RPS // 16, _dz, None)
        pltpu.sync_copy(dz_v, dacc.at[pl.ds(s * RPS, RPS)])

    plsc.subcore_barrier()

    def _gwait(jj, buf, sem):
        pltpu.make_async_copy(table_hbm.at[src_v.at[jj]], buf, sem).wait()

    def _stage(st, _):
        pltpu.sync_copy(src_hbm.at[c, s, pl.ds(st * CPS, CPS)], src_v)
        pltpu.sync_copy(dst_hbm.at[s, pl.ds(st * CPS, CPS)], dst_v)

        def _lin(jj, buf, sem):
            pltpu.async_copy(table_hbm.at[pl.ds(jj * CH, CH)], buf, sem)

        def _lwait(jj, buf, sem):
            pltpu.make_async_copy(table_hbm.at[pl.ds(jj * CH, CH)], buf,
                                  sem).wait()

        _lin(0, buf0, gsem0)

        def _step(t, _):
            jj = 2 * t
            _lin(jj + 1, buf1, gsem1)
            _lwait(jj, buf0, gsem0)

            @pl.when(jj + 2 < CPS)
            def _():
                _lin(jj + 2, buf0, gsem0)

            _lwait(jj + 1, buf1, gsem1)
            return _

        lax.fori_loop(0, CPS // 2, _step, None)
        return _

    lax.fori_loop(0, NCH // CPS, _stage, None)

    plsc.subcore_barrier()

    pltpu.sync_copy(acc.at[pl.ds(s * RPS, RPS)],
                    out_hbm.at[c, pl.ds(s * RPS, RPS)])
    if with_deg:
        @pl.when(c == 0)
        def _():
            pltpu.sync_copy(dacc.at[pl.ds(s * RPS, RPS)],
                            deg_hbm.at[pl.ds(s * RPS, RPS)])


def _make_sc_agg(with_deg):
    mesh = plsc.VectorSubcoreMesh(core_axis_name="c", subcore_axis_name="s",
                                  num_cores=NC, num_subcores=NS)
    out_type = (jax.ShapeDtypeStruct((NC, N_PAD, H), jnp.float32),)
    scratch = [
        pltpu.VMEM((CPS, CH), jnp.int32),
        pltpu.VMEM((CPS, CH), jnp.int32),
        pltpu.VMEM((CH, H), jnp.float32),
        pltpu.VMEM((CH, H), jnp.float32),
    ]
    if with_deg:
        out_type = out_type + (jax.ShapeDtypeStruct((N_PAD,), jnp.float32),)
        scratch += [
            pltpu.VMEM((CH,), jnp.float32),
            pltpu.VMEM((RPS,), jnp.float32),
        ]
    scratch += [pltpu.VMEM_SHARED((N_PAD, H), jnp.float32)]
    if with_deg:
        scratch += [pltpu.VMEM_SHARED((N_PAD,), jnp.float32)]
    scratch += [pltpu.SemaphoreType.DMA] * (5 if with_deg else 4)
    return pl.kernel(functools.partial(_sc_agg_body, with_deg),
                     out_type=out_type if with_deg else out_type[0],
                     mesh=mesh, scratch_types=scratch)


_sc_agg_deg = _make_sc_agg(True)
_sc_agg = _make_sc_agg(False)


def _pre_body(x_ref, ws_ref, wn_ref, b_ref, a_ref, bb_ref):
    xb = x_ref[...]
    a_ref[...] = (jnp.dot(xb, ws_ref[...], preferred_element_type=jnp.float32)
                  + b_ref[...])
    bf = jnp.dot(xb, wn_ref[...], preferred_element_type=jnp.float32)
    bb_ref[0] = bf[:, :H]
    bb_ref[1] = bf[:, H:]


def _agg_h(a_ref, s_ref, deg_ref):
    rdeg = 1.0 / jnp.maximum(deg_ref[...], 1.0)
    agg = jnp.concatenate([s_ref[0], s_ref[1]], axis=-1) * rdeg
    return jnp.maximum(a_ref[...] + agg, 0.0)


def _mid_body(a1_ref, s_ref, deg_ref, ws_ref, wn_ref, b_ref, a2_ref, bb2_ref):
    h = _agg_h(a1_ref, s_ref, deg_ref)
    a2_ref[...] = (jnp.dot(h, ws_ref[...], preferred_element_type=jnp.float32)
                   + b_ref[...])
    bf = jnp.dot(h, wn_ref[...], preferred_element_type=jnp.float32)
    bb2_ref[0] = bf[:, :H]
    bb2_ref[1] = bf[:, H:]


def _post_body(a2_ref, s_ref, deg_ref, out_ref):
    out_ref[...] = _agg_h(a2_ref, s_ref, deg_ref)


_W_SPEC = pl.BlockSpec((F, F), lambda i: (0, 0))
_B_SPEC = pl.BlockSpec((1, F), lambda i: (0, 0))
_ROW_SPEC = pl.BlockSpec((BLK, F), lambda i: (i, 0))
_SPLIT_SPEC = pl.BlockSpec((NC, BLK, H), lambda i: (0, i, 0))
_DEG_SPEC = pl.BlockSpec((BLK, 1), lambda i: (i, 0))

_pre = pl.pallas_call(
    _pre_body,
    grid=(N // BLK,),
    in_specs=[_ROW_SPEC, _W_SPEC, _W_SPEC, _B_SPEC],
    out_specs=[_ROW_SPEC, _SPLIT_SPEC],
    out_shape=[jax.ShapeDtypeStruct((N, F), jnp.float32),
               jax.ShapeDtypeStruct((NC, N, H), jnp.float32)],
)

_mid = pl.pallas_call(
    _mid_body,
    grid=(N // BLK,),
    in_specs=[_ROW_SPEC, _SPLIT_SPEC, _DEG_SPEC, _W_SPEC, _W_SPEC, _B_SPEC],
    out_specs=[_ROW_SPEC, _SPLIT_SPEC],
    out_shape=[jax.ShapeDtypeStruct((N, F), jnp.float32),
               jax.ShapeDtypeStruct((NC, N, H), jnp.float32)],
)

_post = pl.pallas_call(
    _post_body,
    grid=(N // BLK,),
    in_specs=[_ROW_SPEC, _SPLIT_SPEC, _DEG_SPEC],
    out_specs=_ROW_SPEC,
    out_shape=jax.ShapeDtypeStruct((N, F), jnp.float32),
)


def kernel(x, edge_index, W1_self, W1_neigh, b1, W2_self, W2_neigh, b2):
    src = edge_index[0].astype(jnp.int32)
    dst = edge_index[1].astype(jnp.int32)

    pad_n = E_PAD - E
    pad_ids = lax.iota(jnp.int32, pad_n)
    src_p = jnp.concatenate([src, pad_ids % 128])
    dst_p = jnp.concatenate([dst, N + (pad_ids % (N_PAD - N))])
    src2 = jnp.stack([src_p, src_p + N]).reshape(NC, NS, NCH, CH)
    dst3 = dst_p.reshape(NS, NCH, CH)

    a1, bmat1 = _pre(x, W1_self, W1_neigh, b1.reshape(1, F))
    s1, degm = _sc_agg_deg(src2, dst3, bmat1.reshape(NC * N, H))
    deg2 = degm[:N].reshape(N, 1)
    a2, bmat2 = _mid(a1, s1, deg2, W2_self, W2_neigh, b2.reshape(1, F))
    s2 = _sc_agg(src2, dst3, bmat2.reshape(NC * N, H))
    return _post(a2, s2, deg2)


# speedup vs baseline: 1.0039x; 1.0039x over previous
import functools

import jax
import jax.numpy as jnp
from jax import lax
from jax.experimental import pallas as pl
from jax.experimental.pallas import tpu as pltpu
from jax.experimental.pallas import tpu_sc as plsc

N = 10000
E = 160000
F = 256
H = 128
NC = 2
NS = 16
CH = 128
GS = 64
EPW = 10240
E_PAD = NS * EPW
NCH = EPW // CH
CPS = 40
N_PAD = 10240
RPS = N_PAD // NS
BLK = 1000


def _sc_agg_body(with_deg, *refs):
    if with_deg:
        (src_hbm, dst_hbm, table_hbm, out_hbm, deg_hbm,
         src_v, dst_v, buf0, buf1, ones_v, dz_v, acc, dacc,
         gsem0, gsem1, ssem0, ssem1, dsem) = refs
    else:
        (src_hbm, dst_hbm, table_hbm, out_hbm,
         src_v, dst_v, buf0, buf1, acc,
         gsem0, gsem1, ssem0, ssem1) = refs

    c = lax.axis_index("c")
    s = lax.axis_index("s")

    zero16 = jnp.zeros((16,), jnp.float32)

    def _zb(i, _):
        buf0[i // 8, pl.ds((i % 8) * 16, 16)] = zero16
        return _

    lax.fori_loop(0, (CH * H) // 16, _zb, None)
    for k in range(RPS // CH):
        pltpu.sync_copy(buf0, acc.at[pl.ds(s * RPS + k * CH, CH)])

    if with_deg:
        one16 = jnp.ones((16,), jnp.float32)

        def _ob(i, _):
            ones_v[pl.ds(i * 16, 16)] = one16
            return _

        lax.fori_loop(0, CH // 16, _ob, None)

        def _dz(i, _):
            dz_v[pl.ds(i * 16, 16)] = zero16
            return _

        lax.fori_loop(0, RPS // 16, _dz, None)
        pltpu.sync_copy(dz_v, dacc.at[pl.ds(s * RPS, RPS)])

    plsc.subcore_barrier()

    def _gstart(jj, buf, sem):
        for g in range(CH // GS):
            pltpu.async_copy(table_hbm.at[src_v.at[jj, pl.ds(g * GS, GS)]],
                             buf.at[pl.ds(g * GS, GS)], sem)

    def _gwait(jj, buf, sem):
        pltpu.make_async_copy(table_hbm.at[src_v.at[jj]], buf, sem).wait()

    def _swait(jj, buf, sem):
        pltpu.make_async_copy(buf, acc.at[dst_v.at[jj]], sem).wait()

    def _stage(st, _):
        pltpu.sync_copy(src_hbm.at[c, s, pl.ds(st * CPS, CPS)], src_v)
        pltpu.sync_copy(dst_hbm.at[s, pl.ds(st * CPS, CPS)], dst_v)

        _gstart(0, buf0, gsem0)

        def _step(t, _):
            jj = 2 * t

            @pl.when(jj > 0)
            def _():
                _swait(jj - 1, buf1, ssem1)

            _gstart(jj + 1, buf1, gsem1)
            _gwait(jj, buf0, gsem0)
            pltpu.async_copy(buf0, acc.at[dst_v.at[jj]], ssem0, add=True)
            if with_deg:
                pltpu.async_copy(ones_v, dacc.at[dst_v.at[jj]], dsem,
                                 add=True)

            @pl.when(jj + 2 < CPS)
            def _():
                _swait(jj, buf0, ssem0)
                _gstart(jj + 2, buf0, gsem0)

            _gwait(jj + 1, buf1, gsem1)
            pltpu.async_copy(buf1, acc.at[dst_v.at[jj + 1]], ssem1, add=True)
            if with_deg:
                pltpu.async_copy(ones_v, dacc.at[dst_v.at[jj + 1]], dsem,
                                 add=True)
            return _

        lax.fori_loop(0, CPS // 2, _step, None)
        _swait(CPS - 2, buf0, ssem0)
        _swait(CPS - 1, buf1, ssem1)
        if with_deg:
            def _ddrain(t, _):
                pltpu.make_async_copy(ones_v, dacc.at[dst_v.at[t]],
                                      dsem).wait()
                return _

            lax.fori_loop(0, CPS, _ddrain, None)
        return _

    lax.fori_loop(0, NCH // CPS, _stage, None)

    plsc.subcore_barrier()

    pltpu.sync_copy(acc.at[pl.ds(s * RPS, RPS)],
                    out_hbm.at[c, pl.ds(s * RPS, RPS)])
    if with_deg:
        @pl.when(c == 0)
        def _():
            pltpu.sync_copy(dacc.at[pl.ds(s * RPS, RPS)],
                            deg_hbm.at[pl.ds(s * RPS, RPS)])


def _make_sc_agg(with_deg):
    mesh = plsc.VectorSubcoreMesh(core_axis_name="c", subcore_axis_name="s",
                                  num_cores=NC, num_subcores=NS)
    out_type = (jax.ShapeDtypeStruct((NC, N_PAD, H), jnp.float32),)
    scratch = [
        pltpu.VMEM((CPS, CH), jnp.int32),
        pltpu.VMEM((CPS, CH), jnp.int32),
        pltpu.VMEM((CH, H), jnp.float32),
        pltpu.VMEM((CH, H), jnp.float32),
    ]
    if with_deg:
        out_type = out_type + (jax.ShapeDtypeStruct((N_PAD,), jnp.float32),)
        scratch += [
            pltpu.VMEM((CH,), jnp.float32),
            pltpu.VMEM((RPS,), jnp.float32),
        ]
    scratch += [pltpu.VMEM_SHARED((N_PAD, H), jnp.float32)]
    if with_deg:
        scratch += [pltpu.VMEM_SHARED((N_PAD,), jnp.float32)]
    scratch += [pltpu.SemaphoreType.DMA] * (5 if with_deg else 4)
    return pl.kernel(functools.partial(_sc_agg_body, with_deg),
                     out_type=out_type if with_deg else out_type[0],
                     mesh=mesh, scratch_types=scratch)


_sc_agg_deg = _make_sc_agg(True)
_sc_agg = _make_sc_agg(False)


def _pre_body(x_ref, ws_ref, wn_ref, b_ref, a_ref, bb_ref):
    xb = x_ref[...]
    a_ref[...] = (jnp.dot(xb, ws_ref[...], preferred_element_type=jnp.float32)
                  + b_ref[...])
    bf = jnp.dot(xb, wn_ref[...], preferred_element_type=jnp.float32)
    bb_ref[0] = bf[:, :H]
    bb_ref[1] = bf[:, H:]


def _agg_h(a_ref, s_ref, deg_ref):
    rdeg = 1.0 / jnp.maximum(deg_ref[...], 1.0)
    agg = jnp.concatenate([s_ref[0], s_ref[1]], axis=-1) * rdeg
    return jnp.maximum(a_ref[...] + agg, 0.0)


def _mid_body(a1_ref, s_ref, deg_ref, ws_ref, wn_ref, b_ref, a2_ref, bb2_ref):
    h = _agg_h(a1_ref, s_ref, deg_ref)
    a2_ref[...] = (jnp.dot(h, ws_ref[...], preferred_element_type=jnp.float32)
                   + b_ref[...])
    bf = jnp.dot(h, wn_ref[...], preferred_element_type=jnp.float32)
    bb2_ref[0] = bf[:, :H]
    bb2_ref[1] = bf[:, H:]


def _post_body(a2_ref, s_ref, deg_ref, out_ref):
    out_ref[...] = _agg_h(a2_ref, s_ref, deg_ref)


_W_SPEC = pl.BlockSpec((F, F), lambda i: (0, 0))
_B_SPEC = pl.BlockSpec((1, F), lambda i: (0, 0))
_ROW_SPEC = pl.BlockSpec((BLK, F), lambda i: (i, 0))
_SPLIT_SPEC = pl.BlockSpec((NC, BLK, H), lambda i: (0, i, 0))
_DEG_SPEC = pl.BlockSpec((BLK, 1), lambda i: (i, 0))

_pre = pl.pallas_call(
    _pre_body,
    grid=(N // BLK,),
    in_specs=[_ROW_SPEC, _W_SPEC, _W_SPEC, _B_SPEC],
    out_specs=[_ROW_SPEC, _SPLIT_SPEC],
    out_shape=[jax.ShapeDtypeStruct((N, F), jnp.float32),
               jax.ShapeDtypeStruct((NC, N, H), jnp.float32)],
)

_mid = pl.pallas_call(
    _mid_body,
    grid=(N // BLK,),
    in_specs=[_ROW_SPEC, _SPLIT_SPEC, _DEG_SPEC, _W_SPEC, _W_SPEC, _B_SPEC],
    out_specs=[_ROW_SPEC, _SPLIT_SPEC],
    out_shape=[jax.ShapeDtypeStruct((N, F), jnp.float32),
               jax.ShapeDtypeStruct((NC, N, H), jnp.float32)],
)

_post = pl.pallas_call(
    _post_body,
    grid=(N // BLK,),
    in_specs=[_ROW_SPEC, _SPLIT_SPEC, _DEG_SPEC],
    out_specs=_ROW_SPEC,
    out_shape=jax.ShapeDtypeStruct((N, F), jnp.float32),
)


def kernel(x, edge_index, W1_self, W1_neigh, b1, W2_self, W2_neigh, b2):
    src = edge_index[0].astype(jnp.int32)
    dst = edge_index[1].astype(jnp.int32)

    pad_n = E_PAD - E
    pad_ids = lax.iota(jnp.int32, pad_n)
    src_p = jnp.concatenate([src, pad_ids % 128])
    dst_p = jnp.concatenate([dst, N + (pad_ids % (N_PAD - N))])
    src2 = jnp.stack([src_p, src_p + N]).reshape(NC, NS, NCH, CH)
    dst3 = dst_p.reshape(NS, NCH, CH)

    a1, bmat1 = _pre(x, W1_self, W1_neigh, b1.reshape(1, F))
    s1, degm = _sc_agg_deg(src2, dst3, bmat1.reshape(NC * N, H))
    deg2 = degm[:N].reshape(N, 1)
    a2, bmat2 = _mid(a1, s1, deg2, W2_self, W2_neigh, b2.reshape(1, F))
    s2 = _sc_agg(src2, dst3, bmat2.reshape(NC * N, H))
    return _post(a2, s2, deg2)


# speedup vs baseline: 1.0487x; 1.0445x over previous
import jax
import jax.numpy as jnp
from jax import lax
from jax.experimental import pallas as pl
from jax.experimental.pallas import tpu as pltpu
from jax.experimental.pallas import tpu_sc as plsc

N = 10000
E = 160000
F = 256
H = 128
NC = 2
NS = 16
CH = 128
EPW = 10240
E_PAD = NS * EPW
NCH = EPW // CH
CPS = 40
N_PAD = 10240
RPS = N_PAD // NS
BLK = 1000

_MESH = plsc.VectorSubcoreMesh(core_axis_name="c", subcore_axis_name="s",
                               num_cores=NC, num_subcores=NS)


def _sc_agg_body(src_hbm, dst_hbm, table_hbm, out_hbm,
                 src_v, dst_v, buf0, buf1, acc,
                 gsem0, gsem1, ssem0, ssem1):
    c = lax.axis_index("c")
    s = lax.axis_index("s")

    zero16 = jnp.zeros((16,), jnp.float32)

    def _zb(i, _):
        for k in range(8):
            buf0[i, pl.ds(k * 16, 16)] = zero16
        return _

    lax.fori_loop(0, CH, _zb, None)
    for k in range(RPS // CH):
        pltpu.sync_copy(buf0, acc.at[pl.ds(s * RPS + k * CH, CH)])

    plsc.subcore_barrier()

    def _gwait(jj, buf, sem):
        pltpu.make_async_copy(table_hbm.at[src_v.at[jj]], buf, sem).wait()

    def _swait(jj, buf, sem):
        pltpu.make_async_copy(buf, acc.at[dst_v.at[jj]], sem).wait()

    def _stage(st, _):
        pltpu.sync_copy(src_hbm.at[c, s, pl.ds(st * CPS, CPS)], src_v)
        pltpu.sync_copy(dst_hbm.at[s, pl.ds(st * CPS, CPS)], dst_v)

        pltpu.async_copy(table_hbm.at[src_v.at[0]], buf0, gsem0)

        def _step(t, _):
            jj = 2 * t

            @pl.when(jj > 0)
            def _():
                _swait(jj - 1, buf1, ssem1)

            pltpu.async_copy(table_hbm.at[src_v.at[jj + 1]], buf1, gsem1)
            _gwait(jj, buf0, gsem0)
            pltpu.async_copy(buf0, acc.at[dst_v.at[jj]], ssem0, add=True)

            @pl.when(jj + 2 < CPS)
            def _():
                _swait(jj, buf0, ssem0)
                pltpu.async_copy(table_hbm.at[src_v.at[jj + 2]], buf0, gsem0)

            _gwait(jj + 1, buf1, gsem1)
            pltpu.async_copy(buf1, acc.at[dst_v.at[jj + 1]], ssem1, add=True)
            return _

        lax.fori_loop(0, CPS // 2, _step, None)
        _swait(CPS - 2, buf0, ssem0)
        _swait(CPS - 1, buf1, ssem1)
        return _

    lax.fori_loop(0, NCH // CPS, _stage, None)

    plsc.subcore_barrier()

    pltpu.sync_copy(acc.at[pl.ds(s * RPS, RPS)],
                    out_hbm.at[c, pl.ds(s * RPS, RPS)])


_sc_agg = pl.kernel(
    _sc_agg_body,
    out_type=jax.ShapeDtypeStruct((NC, N_PAD, H), jnp.float32),
    mesh=_MESH,
    scratch_types=[
        pltpu.VMEM((CPS, CH), jnp.int32),
        pltpu.VMEM((CPS, CH), jnp.int32),
        pltpu.VMEM((CH, H), jnp.float32),
        pltpu.VMEM((CH, H), jnp.float32),
        pltpu.VMEM_SHARED((N_PAD, H), jnp.float32),
        pltpu.SemaphoreType.DMA,
        pltpu.SemaphoreType.DMA,
        pltpu.SemaphoreType.DMA,
        pltpu.SemaphoreType.DMA,
    ],
)


def _sc_deg_body(dst_hbm, deg_hbm, dst_v, ones_v, dz_v, dacc, dsem):
    c = lax.axis_index("c")
    s = lax.axis_index("s")

    one16 = jnp.ones((16,), jnp.float32)
    zero16 = jnp.zeros((16,), jnp.float32)

    def _ob(i, _):
        ones_v[pl.ds(i * 16, 16)] = one16
        return _

    lax.fori_loop(0, CH // 16, _ob, None)

    def _dz(i, _):
        dz_v[pl.ds(i * 16, 16)] = zero16
        return _

    lax.fori_loop(0, RPS // 16, _dz, None)
    pltpu.sync_copy(dz_v, dacc.at[pl.ds(s * RPS, RPS)])
    plsc.subcore_barrier()

    pltpu.sync_copy(dst_hbm.at[s, pl.ds(c * CPS, CPS)], dst_v)

    def _step(jj, _):
        pltpu.async_copy(ones_v, dacc.at[dst_v.at[jj]], dsem, add=True)
        return _

    lax.fori_loop(0, CPS, _step, None)

    def _ddrain(jj, _):
        pltpu.make_async_copy(ones_v, dacc.at[dst_v.at[jj]], dsem).wait()
        return _

    lax.fori_loop(0, CPS, _ddrain, None)
    plsc.subcore_barrier()

    pltpu.sync_copy(dacc.at[pl.ds(s * RPS, RPS)],
                    deg_hbm.at[c, pl.ds(s * RPS, RPS)])


def _sc_deg(dst_w):
    return pl.kernel(
        _sc_deg_body,
        out_type=jax.ShapeDtypeStruct((NC, N_PAD), jnp.float32),
        mesh=_MESH,
        scratch_types=[
            pltpu.VMEM((CPS, CH), jnp.int32),
            pltpu.VMEM((CH,), jnp.float32),
            pltpu.VMEM((RPS,), jnp.float32),
            pltpu.VMEM_SHARED((N_PAD,), jnp.float32),
            pltpu.SemaphoreType.DMA,
        ],
    )(dst_w)


def _preB_body(x_ref, wn_ref, bb_ref):
    bf = jnp.dot(x_ref[...], wn_ref[...], preferred_element_type=jnp.float32)
    bb_ref[0] = bf[:, :H]
    bb_ref[1] = bf[:, H:]


def _preA_body(x_ref, ws_ref, b_ref, a_ref):
    a_ref[...] = (jnp.dot(x_ref[...], ws_ref[...],
                          preferred_element_type=jnp.float32) + b_ref[...])


def _agg_h(a_ref, s_ref, deg_ref):
    rdeg = 1.0 / jnp.maximum(deg_ref[...], 1.0)
    agg = jnp.concatenate([s_ref[0], s_ref[1]], axis=-1) * rdeg
    return jnp.maximum(a_ref[...] + agg, 0.0)


def _midB_body(a1_ref, s_ref, deg_ref, wn_ref, bb2_ref):
    h = _agg_h(a1_ref, s_ref, deg_ref)
    bf = jnp.dot(h, wn_ref[...], preferred_element_type=jnp.float32)
    bb2_ref[0] = bf[:, :H]
    bb2_ref[1] = bf[:, H:]


def _midA_body(a1_ref, s_ref, deg_ref, ws_ref, b_ref, a2_ref):
    h = _agg_h(a1_ref, s_ref, deg_ref)
    a2_ref[...] = (jnp.dot(h, ws_ref[...],
                           preferred_element_type=jnp.float32) + b_ref[...])


def _post_body(a2_ref, s_ref, deg_ref, out_ref):
    out_ref[...] = _agg_h(a2_ref, s_ref, deg_ref)


_W_SPEC = pl.BlockSpec((F, F), lambda i: (0, 0))
_B_SPEC = pl.BlockSpec((1, F), lambda i: (0, 0))
_ROW_SPEC = pl.BlockSpec((BLK, F), lambda i: (i, 0))
_SPLIT_SPEC = pl.BlockSpec((NC, BLK, H), lambda i: (0, i, 0))
_DEG_SPEC = pl.BlockSpec((BLK, 1), lambda i: (i, 0))

_ROW_SHAPE = jax.ShapeDtypeStruct((N, F), jnp.float32)
_SPLIT_SHAPE = jax.ShapeDtypeStruct((NC, N, H), jnp.float32)

_preB = pl.pallas_call(
    _preB_body, grid=(N // BLK,),
    in_specs=[_ROW_SPEC, _W_SPEC],
    out_specs=_SPLIT_SPEC, out_shape=_SPLIT_SHAPE,
)

_preA = pl.pallas_call(
    _preA_body, grid=(N // BLK,),
    in_specs=[_ROW_SPEC, _W_SPEC, _B_SPEC],
    out_specs=_ROW_SPEC, out_shape=_ROW_SHAPE,
)

_midB = pl.pallas_call(
    _midB_body, grid=(N // BLK,),
    in_specs=[_ROW_SPEC, _SPLIT_SPEC, _DEG_SPEC, _W_SPEC],
    out_specs=_SPLIT_SPEC, out_shape=_SPLIT_SHAPE,
)

_midA = pl.pallas_call(
    _midA_body, grid=(N // BLK,),
    in_specs=[_ROW_SPEC, _SPLIT_SPEC, _DEG_SPEC, _W_SPEC, _B_SPEC],
    out_specs=_ROW_SPEC, out_shape=_ROW_SHAPE,
)

_post = pl.pallas_call(
    _post_body, grid=(N // BLK,),
    in_specs=[_ROW_SPEC, _SPLIT_SPEC, _DEG_SPEC],
    out_specs=_ROW_SPEC, out_shape=_ROW_SHAPE,
)


def kernel(x, edge_index, W1_self, W1_neigh, b1, W2_self, W2_neigh, b2):
    src = edge_index[0].astype(jnp.int32)
    dst = edge_index[1].astype(jnp.int32)

    pad_n = E_PAD - E
    pad_ids = lax.iota(jnp.int32, pad_n)
    src_p = jnp.concatenate([src, pad_ids % 128])
    dst_p = jnp.concatenate([dst, N + (pad_ids % (N_PAD - N))])
    src2 = jnp.stack([src_p, src_p + N]).reshape(NC, NS, NCH, CH)
    dst3 = dst_p.reshape(NS, NCH, CH)

    degm = _sc_deg(dst3)
    deg2 = (degm[0, :N] + degm[1, :N]).reshape(N, 1)

    bmat1 = _preB(x, W1_neigh)
    s1 = _sc_agg(src2, dst3, bmat1.reshape(NC * N, H))
    a1 = _preA(x, W1_self, b1.reshape(1, F))

    bmat2 = _midB(a1, s1, deg2, W2_neigh)
    s2 = _sc_agg(src2, dst3, bmat2.reshape(NC * N, H))
    a2 = _midA(a1, s1, deg2, W2_self, b2.reshape(1, F))

    return _post(a2, s2, deg2)


# speedup vs baseline: 1.0541x; 1.0052x over previous
import jax
import jax.numpy as jnp
import numpy as np
from jax import lax
from jax.experimental import pallas as pl
from jax.experimental.pallas import tpu as pltpu
from jax.experimental.pallas import tpu_sc as plsc

N = 10000
E = 160000
F = 256
H = 128
NC = 2
NS = 16
CH = 128
EPW = 10240
E_PAD = NS * EPW
NCH = EPW // CH
CPS = 40
N_PAD = 10240
RPS = N_PAD // NS
BLK = 1000

_MESH = plsc.VectorSubcoreMesh(core_axis_name="c", subcore_axis_name="s",
                               num_cores=NC, num_subcores=NS)


def _sc_agg_body(src_hbm, dst_hbm, table_hbm, out_hbm,
                 src_v, dst_v, buf0, buf1, acc,
                 gsem0, gsem1, ssem0, ssem1):
    c = lax.axis_index("c")
    s = lax.axis_index("s")

    zero16 = jnp.zeros((16,), jnp.float32)

    def _zb(i, _):
        for k in range(8):
            buf0[i, pl.ds(k * 16, 16)] = zero16
        return _

    lax.fori_loop(0, CH, _zb, None)
    for k in range(RPS // CH):
        pltpu.sync_copy(buf0, acc.at[pl.ds(s * RPS + k * CH, CH)])

    plsc.subcore_barrier()

    def _gwait(jj, buf, sem):
        pltpu.make_async_copy(table_hbm.at[src_v.at[jj]], buf, sem).wait()

    def _swait(jj, buf, sem):
        pltpu.make_async_copy(buf, acc.at[dst_v.at[jj]], sem).wait()

    def _stage(st, _):
        pltpu.sync_copy(src_hbm.at[c, s, pl.ds(st * CPS, CPS)], src_v)
        pltpu.sync_copy(dst_hbm.at[s, pl.ds(st * CPS, CPS)], dst_v)

        pltpu.async_copy(table_hbm.at[src_v.at[0]], buf0, gsem0)

        def _step(t, _):
            jj = 2 * t

            @pl.when(jj > 0)
            def _():
                _swait(jj - 1, buf1, ssem1)

            pltpu.async_copy(table_hbm.at[src_v.at[jj + 1]], buf1, gsem1)
            _gwait(jj, buf0, gsem0)
            pltpu.async_copy(buf0, acc.at[dst_v.at[jj]], ssem0, add=True)

            @pl.when(jj + 2 < CPS)
            def _():
                _swait(jj, buf0, ssem0)
                pltpu.async_copy(table_hbm.at[src_v.at[jj + 2]], buf0, gsem0)

            _gwait(jj + 1, buf1, gsem1)
            pltpu.async_copy(buf1, acc.at[dst_v.at[jj + 1]], ssem1, add=True)
            return _

        lax.fori_loop(0, CPS // 2, _step, None)
        _swait(CPS - 2, buf0, ssem0)
        _swait(CPS - 1, buf1, ssem1)
        return _

    lax.fori_loop(0, NCH // CPS, _stage, None)

    plsc.subcore_barrier()

    pltpu.sync_copy(acc.at[pl.ds(s * RPS, RPS)],
                    out_hbm.at[c, pl.ds(s * RPS, RPS)])


_sc_agg = pl.kernel(
    _sc_agg_body,
    out_type=jax.ShapeDtypeStruct((NC, N_PAD, H), jnp.float32),
    mesh=_MESH,
    scratch_types=[
        pltpu.VMEM((CPS, CH), jnp.int32),
        pltpu.VMEM((CPS, CH), jnp.int32),
        pltpu.VMEM((CH, H), jnp.float32),
        pltpu.VMEM((CH, H), jnp.float32),
        pltpu.VMEM_SHARED((N_PAD, H), jnp.float32),
        pltpu.SemaphoreType.DMA,
        pltpu.SemaphoreType.DMA,
        pltpu.SemaphoreType.DMA,
        pltpu.SemaphoreType.DMA,
    ],
)


def _sc_deg_body(dst_hbm, deg_hbm, dst_v, ones_v, dz_v, dacc, dsem):
    c = lax.axis_index("c")
    s = lax.axis_index("s")

    one16 = jnp.ones((16,), jnp.float32)
    zero16 = jnp.zeros((16,), jnp.float32)

    def _ob(i, _):
        ones_v[pl.ds(i * 16, 16)] = one16
        return _

    lax.fori_loop(0, CH // 16, _ob, None)

    def _dz(i, _):
        dz_v[pl.ds(i * 16, 16)] = zero16
        return _

    lax.fori_loop(0, RPS // 16, _dz, None)
    pltpu.sync_copy(dz_v, dacc.at[pl.ds(s * RPS, RPS)])
    plsc.subcore_barrier()

    pltpu.sync_copy(dst_hbm.at[s, pl.ds(c * CPS, CPS)], dst_v)

    def _step(jj, _):
        pltpu.async_copy(ones_v, dacc.at[dst_v.at[jj]], dsem, add=True)
        return _

    lax.fori_loop(0, CPS, _step, None)

    def _ddrain(jj, _):
        pltpu.make_async_copy(ones_v, dacc.at[dst_v.at[jj]], dsem).wait()
        return _

    lax.fori_loop(0, CPS, _ddrain, None)
    plsc.subcore_barrier()

    pltpu.sync_copy(dacc.at[pl.ds(s * RPS, RPS)],
                    deg_hbm.at[c, pl.ds(s * RPS, RPS)])


def _sc_deg(dst_w):
    return pl.kernel(
        _sc_deg_body,
        out_type=jax.ShapeDtypeStruct((NC, N_PAD), jnp.float32),
        mesh=_MESH,
        scratch_types=[
            pltpu.VMEM((CPS, CH), jnp.int32),
            pltpu.VMEM((CH,), jnp.float32),
            pltpu.VMEM((RPS,), jnp.float32),
            pltpu.VMEM_SHARED((N_PAD,), jnp.float32),
            pltpu.SemaphoreType.DMA,
        ],
    )(dst_w)


def _preB_body(x_ref, wn_ref, bb_ref):
    bf = jnp.dot(x_ref[...], wn_ref[...], preferred_element_type=jnp.float32)
    bb_ref[0] = bf[:, :H]
    bb_ref[1] = bf[:, H:]


def _preA_body(x_ref, ws_ref, b_ref, a_ref):
    a_ref[...] = (jnp.dot(x_ref[...], ws_ref[...],
                          preferred_element_type=jnp.float32) + b_ref[...])


def _agg_h(a_ref, s_ref, deg_ref):
    rdeg = 1.0 / jnp.maximum(deg_ref[...], 1.0)
    agg = jnp.concatenate([s_ref[0], s_ref[1]], axis=-1) * rdeg
    return jnp.maximum(a_ref[...] + agg, 0.0)


def _midB_body(a1_ref, s_ref, deg_ref, wn_ref, bb2_ref):
    h = _agg_h(a1_ref, s_ref, deg_ref)
    bf = jnp.dot(h, wn_ref[...], preferred_element_type=jnp.float32)
    bb2_ref[0] = bf[:, :H]
    bb2_ref[1] = bf[:, H:]


def _midA_body(a1_ref, s_ref, deg_ref, ws_ref, b_ref, a2_ref):
    h = _agg_h(a1_ref, s_ref, deg_ref)
    a2_ref[...] = (jnp.dot(h, ws_ref[...],
                           preferred_element_type=jnp.float32) + b_ref[...])


def _post_body(a2_ref, s_ref, deg_ref, out_ref):
    out_ref[...] = _agg_h(a2_ref, s_ref, deg_ref)


_W_SPEC = pl.BlockSpec((F, F), lambda i: (0, 0))
_B_SPEC = pl.BlockSpec((1, F), lambda i: (0, 0))
_ROW_SPEC = pl.BlockSpec((BLK, F), lambda i: (i, 0))
_SPLIT_SPEC = pl.BlockSpec((NC, BLK, H), lambda i: (0, i, 0))
_DEG_SPEC = pl.BlockSpec((BLK, 1), lambda i: (i, 0))

_ROW_SHAPE = jax.ShapeDtypeStruct((N, F), jnp.float32)
_SPLIT_SHAPE = jax.ShapeDtypeStruct((NC, N, H), jnp.float32)

_preB = pl.pallas_call(
    _preB_body, grid=(N // BLK,),
    in_specs=[_ROW_SPEC, _W_SPEC],
    out_specs=_SPLIT_SPEC, out_shape=_SPLIT_SHAPE,
)

_preA = pl.pallas_call(
    _preA_body, grid=(N // BLK,),
    in_specs=[_ROW_SPEC, _W_SPEC, _B_SPEC],
    out_specs=_ROW_SPEC, out_shape=_ROW_SHAPE,
)

_midB = pl.pallas_call(
    _midB_body, grid=(N // BLK,),
    in_specs=[_ROW_SPEC, _SPLIT_SPEC, _DEG_SPEC, _W_SPEC],
    out_specs=_SPLIT_SPEC, out_shape=_SPLIT_SHAPE,
)

_midA = pl.pallas_call(
    _midA_body, grid=(N // BLK,),
    in_specs=[_ROW_SPEC, _SPLIT_SPEC, _DEG_SPEC, _W_SPEC, _B_SPEC],
    out_specs=_ROW_SPEC, out_shape=_ROW_SHAPE,
)

_post = pl.pallas_call(
    _post_body, grid=(N // BLK,),
    in_specs=[_ROW_SPEC, _SPLIT_SPEC, _DEG_SPEC],
    out_specs=_ROW_SPEC, out_shape=_ROW_SHAPE,
)


_PAD_N = E_PAD - E
_PAD_SRC = np.arange(_PAD_N, dtype=np.int32).reshape(-1, CH) % 128
_PAD_DST = N + np.arange(_PAD_N, dtype=np.int32).reshape(-1, CH) % (N_PAD - N)
_CORE_OFF = np.arange(NC, dtype=np.int32).reshape(NC, 1, 1) * N


def kernel(x, edge_index, W1_self, W1_neigh, b1, W2_self, W2_neigh, b2):
    ei = edge_index.astype(jnp.int32).reshape(2, E // CH, CH)
    src_p = jnp.concatenate([ei[0], jnp.asarray(_PAD_SRC)], axis=0)
    dst_p = jnp.concatenate([ei[1], jnp.asarray(_PAD_DST)], axis=0)
    src2 = (src_p[None] + jnp.asarray(_CORE_OFF)).reshape(NC, NS, NCH, CH)
    dst3 = dst_p.reshape(NS, NCH, CH)

    degm = _sc_deg(dst3)
    deg2 = (degm[0, :N] + degm[1, :N]).reshape(N, 1)

    bmat1 = _preB(x, W1_neigh)
    s1 = _sc_agg(src2, dst3, bmat1.reshape(NC * N, H))
    a1 = _preA(x, W1_self, b1.reshape(1, F))

    bmat2 = _midB(a1, s1, deg2, W2_neigh)
    s2 = _sc_agg(src2, dst3, bmat2.reshape(NC * N, H))
    a2 = _midA(a1, s1, deg2, W2_self, b2.reshape(1, F))

    return _post(a2, s2, deg2)


# speedup vs baseline: 1.0729x; 1.0178x over previous
import jax
import jax.numpy as jnp
import numpy as np
from jax import lax
from jax.experimental import pallas as pl
from jax.experimental.pallas import tpu as pltpu
from jax.experimental.pallas import tpu_sc as plsc

N = 10000
E = 160000
F = 256
H = 128
NC = 2
NS = 16
CH = 128
EPW = 10240
E_PAD = NS * EPW
NCH = EPW // CH
CPS = 40
N_PAD = 10240
RPS = N_PAD // NS
BLK = 1000

_MESH = plsc.VectorSubcoreMesh(core_axis_name="c", subcore_axis_name="s",
                               num_cores=NC, num_subcores=NS)


def _sc_agg_body(edges_hbm, dst_hbm, t0_hbm, t1_hbm, out_hbm,
                 src_v, dst_v, buf0, buf1, acc,
                 gsem0, gsem1, ssem0, ssem1):
    c = lax.axis_index("c")
    s = lax.axis_index("s")

    zero16 = jnp.zeros((16,), jnp.float32)

    def _zb(i, _):
        for k in range(8):
            buf0[i, pl.ds(k * 16, 16)] = zero16
        return _

    lax.fori_loop(0, CH, _zb, None)
    for k in range(RPS // CH):
        pltpu.sync_copy(buf0, acc.at[pl.ds(s * RPS + k * CH, CH)])

    plsc.subcore_barrier()

    def _pipeline(table_hbm):
        def _gstart(jj, buf, sem):
            pltpu.async_copy(
                table_hbm.at[src_v.at[pl.ds(jj * CH, CH)]], buf, sem)

        def _gwait(jj, buf, sem):
            pltpu.make_async_copy(
                table_hbm.at[src_v.at[pl.ds(jj * CH, CH)]], buf, sem).wait()

        def _swait(jj, buf, sem):
            pltpu.make_async_copy(buf, acc.at[dst_v.at[jj]], sem).wait()

        def _stage(st, _):
            pltpu.sync_copy(
                edges_hbm.at[0, pl.ds(s * EPW + st * CPS * CH, CPS * CH)],
                src_v)
            pltpu.sync_copy(dst_hbm.at[s, pl.ds(st * CPS, CPS)], dst_v)

            _gstart(0, buf0, gsem0)

            def _step(t, _):
                jj = 2 * t

                @pl.when(jj > 0)
                def _():
                    _swait(jj - 1, buf1, ssem1)

                _gstart(jj + 1, buf1, gsem1)
                _gwait(jj, buf0, gsem0)
                pltpu.async_copy(buf0, acc.at[dst_v.at[jj]], ssem0, add=True)

                @pl.when(jj + 2 < CPS)
                def _():
                    _swait(jj, buf0, ssem0)
                    _gstart(jj + 2, buf0, gsem0)

                _gwait(jj + 1, buf1, gsem1)
                pltpu.async_copy(buf1, acc.at[dst_v.at[jj + 1]], ssem1,
                                 add=True)
                return _

            lax.fori_loop(0, CPS // 2, _step, None)
            _swait(CPS - 2, buf0, ssem0)
            _swait(CPS - 1, buf1, ssem1)
            return _

        lax.fori_loop(0, NCH // CPS, _stage, None)

    @pl.when(c == 0)
    def _():
        _pipeline(t0_hbm)

    @pl.when(c == 1)
    def _():
        _pipeline(t1_hbm)

    plsc.subcore_barrier()

    pltpu.sync_copy(acc.at[pl.ds(s * RPS, RPS)],
                    out_hbm.at[c, pl.ds(s * RPS, RPS)])


_sc_agg = pl.kernel(
    _sc_agg_body,
    out_type=jax.ShapeDtypeStruct((NC, N_PAD, H), jnp.float32),
    mesh=_MESH,
    scratch_types=[
        pltpu.VMEM((CPS * CH,), jnp.int32),
        pltpu.VMEM((CPS, CH), jnp.int32),
        pltpu.VMEM((CH, H), jnp.float32),
        pltpu.VMEM((CH, H), jnp.float32),
        pltpu.VMEM_SHARED((N_PAD, H), jnp.float32),
        pltpu.SemaphoreType.DMA,
        pltpu.SemaphoreType.DMA,
        pltpu.SemaphoreType.DMA,
        pltpu.SemaphoreType.DMA,
    ],
)


def _sc_deg_body(dst_hbm, deg_hbm, dst_v, ones_v, dz_v, dacc, dsem):
    c = lax.axis_index("c")
    s = lax.axis_index("s")

    one16 = jnp.ones((16,), jnp.float32)
    zero16 = jnp.zeros((16,), jnp.float32)

    def _ob(i, _):
        ones_v[pl.ds(i * 16, 16)] = one16
        return _

    lax.fori_loop(0, CH // 16, _ob, None)

    def _dz(i, _):
        dz_v[pl.ds(i * 16, 16)] = zero16
        return _

    lax.fori_loop(0, RPS // 16, _dz, None)
    pltpu.sync_copy(dz_v, dacc.at[pl.ds(s * RPS, RPS)])
    plsc.subcore_barrier()

    pltpu.sync_copy(dst_hbm.at[s, pl.ds(c * CPS, CPS)], dst_v)

    def _step(jj, _):
        pltpu.async_copy(ones_v, dacc.at[dst_v.at[jj]], dsem, add=True)
        return _

    lax.fori_loop(0, CPS, _step, None)

    def _ddrain(jj, _):
        pltpu.make_async_copy(ones_v, dacc.at[dst_v.at[jj]], dsem).wait()
        return _

    lax.fori_loop(0, CPS, _ddrain, None)
    plsc.subcore_barrier()

    pltpu.sync_copy(dacc.at[pl.ds(s * RPS, RPS)],
                    deg_hbm.at[c, pl.ds(s * RPS, RPS)])


def _sc_deg(dst_w):
    return pl.kernel(
        _sc_deg_body,
        out_type=jax.ShapeDtypeStruct((NC, N_PAD), jnp.float32),
        mesh=_MESH,
        scratch_types=[
            pltpu.VMEM((CPS, CH), jnp.int32),
            pltpu.VMEM((CH,), jnp.float32),
            pltpu.VMEM((RPS,), jnp.float32),
            pltpu.VMEM_SHARED((N_PAD,), jnp.float32),
            pltpu.SemaphoreType.DMA,
        ],
    )(dst_w)


def _preB_body(x_ref, wn_ref, b0_ref, b1_ref):
    bf = jnp.dot(x_ref[...], wn_ref[...], preferred_element_type=jnp.float32)
    b0_ref[...] = bf[:, :H]
    b1_ref[...] = bf[:, H:]


def _preA_body(x_ref, ws_ref, b_ref, a_ref):
    a_ref[...] = (jnp.dot(x_ref[...], ws_ref[...],
                          preferred_element_type=jnp.float32) + b_ref[...])


def _agg_h(a_ref, s_ref, deg_ref):
    rdeg = 1.0 / jnp.maximum(deg_ref[...], 1.0)
    agg = jnp.concatenate([s_ref[0], s_ref[1]], axis=-1) * rdeg
    return jnp.maximum(a_ref[...] + agg, 0.0)


def _midB_body(a1_ref, s_ref, deg_ref, wn_ref, b0_ref, b1_ref):
    h = _agg_h(a1_ref, s_ref, deg_ref)
    bf = jnp.dot(h, wn_ref[...], preferred_element_type=jnp.float32)
    b0_ref[...] = bf[:, :H]
    b1_ref[...] = bf[:, H:]


def _midA_body(a1_ref, s_ref, deg_ref, ws_ref, b_ref, a2_ref):
    h = _agg_h(a1_ref, s_ref, deg_ref)
    a2_ref[...] = (jnp.dot(h, ws_ref[...],
                           preferred_element_type=jnp.float32) + b_ref[...])


def _post_body(a2_ref, s_ref, deg_ref, out_ref):
    out_ref[...] = _agg_h(a2_ref, s_ref, deg_ref)


_W_SPEC = pl.BlockSpec((F, F), lambda i: (0, 0))
_B_SPEC = pl.BlockSpec((1, F), lambda i: (0, 0))
_ROW_SPEC = pl.BlockSpec((BLK, F), lambda i: (i, 0))
_SPLIT_SPEC = pl.BlockSpec((NC, BLK, H), lambda i: (0, i, 0))
_DEG_SPEC = pl.BlockSpec((BLK, 1), lambda i: (i, 0))

_ROW_SHAPE = jax.ShapeDtypeStruct((N, F), jnp.float32)
_HALF_SPEC = pl.BlockSpec((BLK, H), lambda i: (i, 0))
_HALF_SHAPE = jax.ShapeDtypeStruct((N, H), jnp.float32)

_preB = pl.pallas_call(
    _preB_body, grid=(N // BLK,),
    in_specs=[_ROW_SPEC, _W_SPEC],
    out_specs=[_HALF_SPEC, _HALF_SPEC],
    out_shape=[_HALF_SHAPE, _HALF_SHAPE],
)

_preA = pl.pallas_call(
    _preA_body, grid=(N // BLK,),
    in_specs=[_ROW_SPEC, _W_SPEC, _B_SPEC],
    out_specs=_ROW_SPEC, out_shape=_ROW_SHAPE,
)

_midB = pl.pallas_call(
    _midB_body, grid=(N // BLK,),
    in_specs=[_ROW_SPEC, _SPLIT_SPEC, _DEG_SPEC, _W_SPEC],
    out_specs=[_HALF_SPEC, _HALF_SPEC],
    out_shape=[_HALF_SHAPE, _HALF_SHAPE],
)

_midA = pl.pallas_call(
    _midA_body, grid=(N // BLK,),
    in_specs=[_ROW_SPEC, _SPLIT_SPEC, _DEG_SPEC, _W_SPEC, _B_SPEC],
    out_specs=_ROW_SPEC, out_shape=_ROW_SHAPE,
)

_post = pl.pallas_call(
    _post_body, grid=(N // BLK,),
    in_specs=[_ROW_SPEC, _SPLIT_SPEC, _DEG_SPEC],
    out_specs=_ROW_SPEC, out_shape=_ROW_SHAPE,
)


_PAD_N = E_PAD - E
_PAD_EDGES = np.stack([
    np.arange(_PAD_N, dtype=np.int32) % 128,
    N + np.arange(_PAD_N, dtype=np.int32) % (N_PAD - N),
])
_PAD_DST2 = _PAD_EDGES[1].reshape(-1, CH)


def kernel(x, edge_index, W1_self, W1_neigh, b1, W2_self, W2_neigh, b2):
    ei = edge_index.astype(jnp.int32)
    epad = jnp.concatenate([ei, jnp.asarray(_PAD_EDGES)], axis=1)
    dst_p = jnp.concatenate([ei.reshape(2, E // CH, CH)[1],
                             jnp.asarray(_PAD_DST2)], axis=0)
    dst3 = dst_p.reshape(NS, NCH, CH)

    degm = _sc_deg(dst3)
    deg2 = (degm[0, :N] + degm[1, :N]).reshape(N, 1)

    b10, b11 = _preB(x, W1_neigh)
    s1 = _sc_agg(epad, dst3, b10, b11)
    a1 = _preA(x, W1_self, b1.reshape(1, F))

    b20, b21 = _midB(a1, s1, deg2, W2_neigh)
    s2 = _sc_agg(epad, dst3, b20, b21)
    a2 = _midA(a1, s1, deg2, W2_self, b2.reshape(1, F))

    return _post(a2, s2, deg2)


# speedup vs baseline: 1.0813x; 1.0078x over previous
import jax
import jax.numpy as jnp
import numpy as np
from jax import lax
from jax.experimental import pallas as pl
from jax.experimental.pallas import tpu as pltpu
from jax.experimental.pallas import tpu_sc as plsc

N = 10000
E = 160000
F = 256
H = 128
NC = 2
NS = 16
CH = 128
EPW = 10240
E_PAD = NS * EPW
NCH = EPW // CH
CPS = 40
N_PAD = 10240
RPS = N_PAD // NS
BLK = 1000
_TAIL_REAL = E - (NS - 1) * EPW - (NCH // CPS - 1) * CPS * CH

_MESH = plsc.VectorSubcoreMesh(core_axis_name="c", subcore_axis_name="s",
                               num_cores=NC, num_subcores=NS)


def _sc_agg_body(edges_hbm, dst_hbm, t0_hbm, t1_hbm, out_hbm,
                 src_v, dst_v, buf0, buf1, acc,
                 gsem0, gsem1, ssem0, ssem1):
    c = lax.axis_index("c")
    s = lax.axis_index("s")

    zero16 = jnp.zeros((16,), jnp.float32)

    def _zb(i, _):
        for k in range(8):
            buf0[i, pl.ds(k * 16, 16)] = zero16
        return _

    lax.fori_loop(0, CH, _zb, None)
    for k in range(RPS // CH):
        pltpu.sync_copy(buf0, acc.at[pl.ds(s * RPS + k * CH, CH)])

    plsc.subcore_barrier()

    def _pipeline(table_hbm):
        def _gstart(jj, buf, sem):
            pltpu.async_copy(
                table_hbm.at[src_v.at[pl.ds(jj * CH, CH)]], buf, sem)

        def _gwait(jj, buf, sem):
            pltpu.make_async_copy(
                table_hbm.at[src_v.at[pl.ds(jj * CH, CH)]], buf, sem).wait()

        def _swait(jj, buf, sem):
            pltpu.make_async_copy(buf, acc.at[dst_v.at[jj]], sem).wait()

        def _stage(st, _):
            base = s * EPW + st * (CPS * CH)
            tail = jnp.logical_and(s == NS - 1, st == (NCH // CPS) - 1)

            @pl.when(jnp.logical_not(tail))
            def _():
                pltpu.sync_copy(edges_hbm.at[0, pl.ds(base, CPS * CH)],
                                src_v)

            @pl.when(tail)
            def _():
                pltpu.sync_copy(edges_hbm.at[0, pl.ds(base, _TAIL_REAL)],
                                src_v.at[pl.ds(0, _TAIL_REAL)])
                iota16 = lax.iota(jnp.int32, 16)

                def _fill(i, _):
                    src_v[pl.ds(_TAIL_REAL + i * 16, 16)] = (
                        iota16 + (i % 8) * 16)
                    return _

                lax.fori_loop(0, (CPS * CH - _TAIL_REAL) // 16, _fill, None)

            pltpu.sync_copy(dst_hbm.at[s, pl.ds(st * CPS, CPS)], dst_v)

            _gstart(0, buf0, gsem0)

            def _step(t, _):
                jj = 2 * t

                @pl.when(jj > 0)
                def _():
                    _swait(jj - 1, buf1, ssem1)

                _gstart(jj + 1, buf1, gsem1)
                _gwait(jj, buf0, gsem0)
                pltpu.async_copy(buf0, acc.at[dst_v.at[jj]], ssem0, add=True)

                @pl.when(jj + 2 < CPS)
                def _():
                    _swait(jj, buf0, ssem0)
                    _gstart(jj + 2, buf0, gsem0)

                _gwait(jj + 1, buf1, gsem1)
                pltpu.async_copy(buf1, acc.at[dst_v.at[jj + 1]], ssem1,
                                 add=True)
                return _

            lax.fori_loop(0, CPS // 2, _step, None)
            _swait(CPS - 2, buf0, ssem0)
            _swait(CPS - 1, buf1, ssem1)
            return _

        lax.fori_loop(0, NCH // CPS, _stage, None)

    @pl.when(c == 0)
    def _():
        _pipeline(t0_hbm)

    @pl.when(c == 1)
    def _():
        _pipeline(t1_hbm)

    plsc.subcore_barrier()

    pltpu.sync_copy(acc.at[pl.ds(s * RPS, RPS)],
                    out_hbm.at[c, pl.ds(s * RPS, RPS)])


_sc_agg = pl.kernel(
    _sc_agg_body,
    out_type=jax.ShapeDtypeStruct((NC, N_PAD, H), jnp.float32),
    mesh=_MESH,
    scratch_types=[
        pltpu.VMEM((CPS * CH,), jnp.int32),
        pltpu.VMEM((CPS, CH), jnp.int32),
        pltpu.VMEM((CH, H), jnp.float32),
        pltpu.VMEM((CH, H), jnp.float32),
        pltpu.VMEM_SHARED((N_PAD, H), jnp.float32),
        pltpu.SemaphoreType.DMA,
        pltpu.SemaphoreType.DMA,
        pltpu.SemaphoreType.DMA,
        pltpu.SemaphoreType.DMA,
    ],
)


def _sc_deg_body(dst_hbm, deg_hbm, dst_v, ones_v, dz_v, dacc, dsem):
    c = lax.axis_index("c")
    s = lax.axis_index("s")

    one16 = jnp.ones((16,), jnp.float32)
    zero16 = jnp.zeros((16,), jnp.float32)

    def _ob(i, _):
        ones_v[pl.ds(i * 16, 16)] = one16
        return _

    lax.fori_loop(0, CH // 16, _ob, None)

    def _dz(i, _):
        dz_v[pl.ds(i * 16, 16)] = zero16
        return _

    lax.fori_loop(0, RPS // 16, _dz, None)
    pltpu.sync_copy(dz_v, dacc.at[pl.ds(s * RPS, RPS)])
    plsc.subcore_barrier()

    pltpu.sync_copy(dst_hbm.at[s, pl.ds(c * CPS, CPS)], dst_v)

    def _step(jj, _):
        pltpu.async_copy(ones_v, dacc.at[dst_v.at[jj]], dsem, add=True)
        return _

    lax.fori_loop(0, CPS, _step, None)

    def _ddrain(jj, _):
        pltpu.make_async_copy(ones_v, dacc.at[dst_v.at[jj]], dsem).wait()
        return _

    lax.fori_loop(0, CPS, _ddrain, None)
    plsc.subcore_barrier()

    pltpu.sync_copy(dacc.at[pl.ds(s * RPS, RPS)],
                    deg_hbm.at[c, pl.ds(s * RPS, RPS)])


def _sc_deg(dst_w):
    return pl.kernel(
        _sc_deg_body,
        out_type=jax.ShapeDtypeStruct((NC, N_PAD), jnp.float32),
        mesh=_MESH,
        scratch_types=[
            pltpu.VMEM((CPS, CH), jnp.int32),
            pltpu.VMEM((CH,), jnp.float32),
            pltpu.VMEM((RPS,), jnp.float32),
            pltpu.VMEM_SHARED((N_PAD,), jnp.float32),
            pltpu.SemaphoreType.DMA,
        ],
    )(dst_w)


def _preB_body(x_ref, wn_ref, b0_ref, b1_ref):
    bf = jnp.dot(x_ref[...], wn_ref[...], preferred_element_type=jnp.float32)
    b0_ref[...] = bf[:, :H]
    b1_ref[...] = bf[:, H:]


def _preA_body(x_ref, ws_ref, b_ref, a_ref):
    a_ref[...] = (jnp.dot(x_ref[...], ws_ref[...],
                          preferred_element_type=jnp.float32) + b_ref[...])


def _agg_h(a_ref, s_ref, deg_ref):
    rdeg = 1.0 / jnp.maximum(deg_ref[...], 1.0)
    agg = jnp.concatenate([s_ref[0], s_ref[1]], axis=-1) * rdeg
    return jnp.maximum(a_ref[...] + agg, 0.0)


def _midB_body(a1_ref, s_ref, deg_ref, wn_ref, b0_ref, b1_ref):
    h = _agg_h(a1_ref, s_ref, deg_ref)
    bf = jnp.dot(h, wn_ref[...], preferred_element_type=jnp.float32)
    b0_ref[...] = bf[:, :H]
    b1_ref[...] = bf[:, H:]


def _midA_body(a1_ref, s_ref, deg_ref, ws_ref, b_ref, a2_ref):
    h = _agg_h(a1_ref, s_ref, deg_ref)
    a2_ref[...] = (jnp.dot(h, ws_ref[...],
                           preferred_element_type=jnp.float32) + b_ref[...])


def _post_body(a2_ref, s_ref, deg_ref, out_ref):
    out_ref[...] = _agg_h(a2_ref, s_ref, deg_ref)


_W_SPEC = pl.BlockSpec((F, F), lambda i: (0, 0))
_B_SPEC = pl.BlockSpec((1, F), lambda i: (0, 0))
_ROW_SPEC = pl.BlockSpec((BLK, F), lambda i: (i, 0))
_SPLIT_SPEC = pl.BlockSpec((NC, BLK, H), lambda i: (0, i, 0))
_DEG_SPEC = pl.BlockSpec((BLK, 1), lambda i: (i, 0))

_ROW_SHAPE = jax.ShapeDtypeStruct((N, F), jnp.float32)
_HALF_SPEC = pl.BlockSpec((BLK, H), lambda i: (i, 0))
_HALF_SHAPE = jax.ShapeDtypeStruct((N, H), jnp.float32)

_preB = pl.pallas_call(
    _preB_body, grid=(N // BLK,),
    in_specs=[_ROW_SPEC, _W_SPEC],
    out_specs=[_HALF_SPEC, _HALF_SPEC],
    out_shape=[_HALF_SHAPE, _HALF_SHAPE],
)

_preA = pl.pallas_call(
    _preA_body, grid=(N // BLK,),
    in_specs=[_ROW_SPEC, _W_SPEC, _B_SPEC],
    out_specs=_ROW_SPEC, out_shape=_ROW_SHAPE,
)

_midB = pl.pallas_call(
    _midB_body, grid=(N // BLK,),
    in_specs=[_ROW_SPEC, _SPLIT_SPEC, _DEG_SPEC, _W_SPEC],
    out_specs=[_HALF_SPEC, _HALF_SPEC],
    out_shape=[_HALF_SHAPE, _HALF_SHAPE],
)

_midA = pl.pallas_call(
    _midA_body, grid=(N // BLK,),
    in_specs=[_ROW_SPEC, _SPLIT_SPEC, _DEG_SPEC, _W_SPEC, _B_SPEC],
    out_specs=_ROW_SPEC, out_shape=_ROW_SHAPE,
)

_post = pl.pallas_call(
    _post_body, grid=(N // BLK,),
    in_specs=[_ROW_SPEC, _SPLIT_SPEC, _DEG_SPEC],
    out_specs=_ROW_SPEC, out_shape=_ROW_SHAPE,
)


_PAD_N = E_PAD - E
_PAD_DST2 = (N + np.arange(_PAD_N, dtype=np.int32)
             % (N_PAD - N)).reshape(-1, CH)


def kernel(x, edge_index, W1_self, W1_neigh, b1, W2_self, W2_neigh, b2):
    ei = edge_index.astype(jnp.int32)
    dst_p = jnp.concatenate([ei.reshape(2, E // CH, CH)[1],
                             jnp.asarray(_PAD_DST2)], axis=0)
    dst3 = dst_p.reshape(NS, NCH, CH)

    degm = _sc_deg(dst3)
    deg2 = (degm[0, :N] + degm[1, :N]).reshape(N, 1)

    b10, b11 = _preB(x, W1_neigh)
    s1 = _sc_agg(ei, dst3, b10, b11)
    a1 = _preA(x, W1_self, b1.reshape(1, F))

    b20, b21 = _midB(a1, s1, deg2, W2_neigh)
    s2 = _sc_agg(ei, dst3, b20, b21)
    a2 = _midA(a1, s1, deg2, W2_self, b2.reshape(1, F))

    return _post(a2, s2, deg2)
